# Initial kernel scaffold; baseline (speedup 1.0000x reference)
import jax
import jax.numpy as jnp
from jax.experimental import pallas as pl


def kernel(window, params):
    raise NotImplementedError("write your pallas kernel here")


# speedup vs baseline: 45.4683x; 45.4683x over previous
import functools

import jax
import jax.numpy as jnp
import numpy as np
from jax.experimental import pallas as pl
from jax.experimental.pallas import tpu as pltpu

_HEADS = 4
_NEG = -1e30


def _attn_expand(a):
    h, dh = a.shape
    eye = jnp.eye(h, dtype=a.dtype)
    return (a[:, :, None] * eye[:, None, :]).reshape(h * dh, h)


def _head_expand(dout):
    return jnp.repeat(jnp.eye(_HEADS, dtype=jnp.float32), dout // _HEADS, axis=1)


def _ln(x, g, b):
    m = jnp.mean(x, axis=-1, keepdims=True)
    v = jnp.mean((x - m) * (x - m), axis=-1, keepdims=True)
    return (x - m) * jax.lax.rsqrt(v + 1e-5) * g + b


def _lrelu(x):
    return jnp.where(x >= 0, x, 0.2 * x)


def _mm(a, b):
    return jnp.dot(a, b, preferred_element_type=jnp.float32)


def _gat(x, W, A_src, A_dst, E, b):
    h = _mm(x, W)
    asrc = _mm(h, A_src)
    adst = _mm(h, A_dst)
    neg = jnp.full((1, _HEADS), _NEG, jnp.float32)
    asrc_m1 = jnp.concatenate([neg, asrc[:-1]], axis=0)
    asrc_p1 = jnp.concatenate([asrc[1:], neg], axis=0)
    e_s = _lrelu(asrc + adst)
    e_m = _lrelu(asrc_m1 + adst)
    e_p = _lrelu(asrc_p1 + adst)
    emax = jnp.maximum(e_s, jnp.maximum(e_m, e_p))
    ex_s = jnp.exp(e_s - emax)
    ex_m = jnp.exp(e_m - emax)
    ex_p = jnp.exp(e_p - emax)
    den = ex_s + ex_m + ex_p + 1e-16
    zrow = jnp.zeros((1, h.shape[1]), jnp.float32)
    h_m1 = jnp.concatenate([zrow, h[:-1]], axis=0)
    h_p1 = jnp.concatenate([h[1:], zrow], axis=0)
    out = (_mm(ex_s / den, E) * h
           + _mm(ex_m / den, E) * h_m1
           + _mm(ex_p / den, E) * h_p1)
    return out + b


def _tx(x, tp):
    n, d = x.shape
    dh = d // _HEADS
    qkv = _mm(x, tp['in_w']) + tp['in_b']
    scale = 1.0 / np.sqrt(dh)
    outs = []
    for k in range(_HEADS):
        q = qkv[:, k * dh:(k + 1) * dh]
        kk = qkv[:, d + k * dh:d + (k + 1) * dh]
        v = qkv[:, 2 * d + k * dh:2 * d + (k + 1) * dh]
        s = jax.lax.dot_general(q, kk, (((1,), (1,)), ((), ())),
                                preferred_element_type=jnp.float32) * scale
        s = s - jnp.max(s, axis=-1, keepdims=True)
        es = jnp.exp(s)
        att = es / jnp.sum(es, axis=-1, keepdims=True)
        outs.append(_mm(att, v))
    o = _mm(jnp.concatenate(outs, axis=1), tp['out_w']) + tp['out_b']
    x = _ln(x + o, tp['ln1_g'], tp['ln1_b'])
    ff = _mm(jnp.maximum(_mm(x, tp['ff1_w']) + tp['ff1_b'], 0.0),
             tp['ff2_w']) + tp['ff2_b']
    return _ln(x + ff, tp['ln2_g'], tp['ln2_b'])


def _fwd_body(treedef, *refs):
    out_x_ref = refs[-2]
    out_logits_ref = refs[-1]
    vals = [r[:] for r in refs[:-2]]
    p = jax.tree_util.tree_unflatten(treedef, vals)

    x = p['window']
    feats = [x]
    for lp in p['enc']:
        g = _gat(x, lp['W'], lp['A_src'], lp['A_dst'], lp['E'], lp['b'])
        x = jnp.maximum(_ln(g, lp['ln_g'], lp['ln_b']), 0.0)
        feats.append(x)
    bx = feats[-1]
    for tp in p['trans']:
        bx = _tx(bx, tp)
    feats[-1] = bx

    gfeat = jnp.mean(bx, axis=0, keepdims=True)
    hcls = jnp.maximum(_mm(gfeat, p['cls1_w']) + p['cls1_b'], 0.0)
    out_logits_ref[:] = _mm(hcls, p['cls2_w']) + p['cls2_b']

    x = bx
    for i, lp in enumerate(p['dec']):
        g = _gat(x, lp['W'], lp['A_src'], lp['A_dst'], lp['E'], lp['b'])
        x = jnp.maximum(_ln(g, lp['ln_g'], lp['ln_b']), 0.0)
        ef = feats[-(i + 2)]
        aligned = jnp.maximum(
            _ln(_mm(ef, lp['al_w']) + lp['al_b'], lp['al_g'], lp['al_be']), 0.0)
        cat = jnp.concatenate([ef, x], axis=-1)
        gate = jax.nn.sigmoid(
            _mm(jnp.maximum(_mm(cat, lp['g1_w']) + lp['g1_b'], 0.0),
                lp['g2_w']) + lp['g2_b'])
        fin = jnp.concatenate([aligned * gate, x], axis=-1)
        x = jnp.maximum(
            _ln(_mm(fin, lp['fu_w']) + lp['fu_b'], lp['fu_g'], lp['fu_be']), 0.0)

    out_x_ref[:] = x.T


def kernel(window, params):
    n, _ = window.shape

    def row(v):
        return v.reshape(1, -1)

    tree = {'window': window, 'enc': [], 'trans': [], 'dec': []}
    for lp in params['enc']:
        dout = lp['W'].shape[1]
        tree['enc'].append({
            'W': lp['W'],
            'A_src': _attn_expand(lp['a_src']),
            'A_dst': _attn_expand(lp['a_dst']),
            'E': _head_expand(dout),
            'b': row(lp['b']),
            'ln_g': row(lp['ln_g']), 'ln_b': row(lp['ln_b'])})
    for tp in params['trans']:
        tree['trans'].append({
            'in_w': tp['in_w'], 'in_b': row(tp['in_b']),
            'out_w': tp['out_w'], 'out_b': row(tp['out_b']),
            'ln1_g': row(tp['ln1_g']), 'ln1_b': row(tp['ln1_b']),
            'ln2_g': row(tp['ln2_g']), 'ln2_b': row(tp['ln2_b']),
            'ff1_w': tp['ff1_w'], 'ff1_b': row(tp['ff1_b']),
            'ff2_w': tp['ff2_w'], 'ff2_b': row(tp['ff2_b'])})
    for lp in params['dec']:
        dout = lp['W'].shape[1]
        tree['dec'].append({
            'W': lp['W'],
            'A_src': _attn_expand(lp['a_src']),
            'A_dst': _attn_expand(lp['a_dst']),
            'E': _head_expand(dout),
            'b': row(lp['b']),
            'ln_g': row(lp['ln_g']), 'ln_b': row(lp['ln_b']),
            'al_w': lp['al_w'], 'al_b': row(lp['al_b']),
            'al_g': row(lp['al_g']), 'al_be': row(lp['al_be']),
            'g1_w': lp['g1_w'], 'g1_b': row(lp['g1_b']),
            'g2_w': lp['g2_w'], 'g2_b': row(lp['g2_b']),
            'fu_w': lp['fu_w'], 'fu_b': row(lp['fu_b']),
            'fu_g': row(lp['fu_g']), 'fu_be': row(lp['fu_be'])})
    tree['cls1_w'] = params['cls1_w']
    tree['cls1_b'] = row(params['cls1_b'])
    tree['cls2_w'] = params['cls2_w']
    tree['cls2_b'] = row(params['cls2_b'])

    flat, treedef = jax.tree_util.tree_flatten(tree)
    out_ch = params['dec'][-1]['W'].shape[1]

    x_t, logits = pl.pallas_call(
        functools.partial(_fwd_body, treedef),
        out_shape=[
            jax.ShapeDtypeStruct((out_ch, n), jnp.float32),
            jax.ShapeDtypeStruct((1, 2), jnp.float32),
        ],
        compiler_params=pltpu.CompilerParams(
            vmem_limit_bytes=128 * 1024 * 1024),
    )(*flat)
    return (x_t, logits.reshape(2))


# speedup vs baseline: 45.5767x; 1.0024x over previous
import functools

import jax
import jax.numpy as jnp
import numpy as np
from jax.experimental import pallas as pl
from jax.experimental.pallas import tpu as pltpu

_HEADS = 4
_NEG = -1e30


def _attn_expand(a):
    h, dh = a.shape
    eye = jnp.eye(h, dtype=a.dtype)
    return (a[:, :, None] * eye[:, None, :]).reshape(h * dh, h)


def _head_expand(dout):
    return jnp.repeat(jnp.eye(_HEADS, dtype=jnp.float32), dout // _HEADS, axis=1)


def _ln(x, g, b):
    m = jnp.mean(x, axis=-1, keepdims=True)
    v = jnp.mean((x - m) * (x - m), axis=-1, keepdims=True)
    return (x - m) * jax.lax.rsqrt(v + 1e-5) * g + b


def _lrelu(x):
    return jnp.where(x >= 0, x, 0.2 * x)


def _mm(a, b):
    return jnp.dot(a, b, preferred_element_type=jnp.float32)


def _mmx(a, b):
    return jnp.dot(a.astype(jnp.bfloat16), b.astype(jnp.bfloat16),
                   preferred_element_type=jnp.float32)


def _gat(x, W, A_src, A_dst, E, b):
    h = _mm(x, W)
    asrc = _mm(h, A_src)
    adst = _mm(h, A_dst)
    neg = jnp.full((1, _HEADS), _NEG, jnp.float32)
    asrc_m1 = jnp.concatenate([neg, asrc[:-1]], axis=0)
    asrc_p1 = jnp.concatenate([asrc[1:], neg], axis=0)
    e_s = _lrelu(asrc + adst)
    e_m = _lrelu(asrc_m1 + adst)
    e_p = _lrelu(asrc_p1 + adst)
    emax = jnp.maximum(e_s, jnp.maximum(e_m, e_p))
    ex_s = jnp.exp(e_s - emax)
    ex_m = jnp.exp(e_m - emax)
    ex_p = jnp.exp(e_p - emax)
    den = ex_s + ex_m + ex_p + 1e-16
    zrow = jnp.zeros((1, h.shape[1]), jnp.float32)
    h_m1 = jnp.concatenate([zrow, h[:-1]], axis=0)
    h_p1 = jnp.concatenate([h[1:], zrow], axis=0)
    out = (_mm(ex_s / den, E) * h
           + _mm(ex_m / den, E) * h_m1
           + _mm(ex_p / den, E) * h_p1)
    return out + b


def _tx(x, tp):
    n, d = x.shape
    dh = d // _HEADS
    qkv = _mmx(x, tp['in_w']) + tp['in_b']
    scale = 1.0 / np.sqrt(dh)
    outs = []
    for k in range(_HEADS):
        q = qkv[:, k * dh:(k + 1) * dh]
        kk = qkv[:, d + k * dh:d + (k + 1) * dh]
        v = qkv[:, 2 * d + k * dh:2 * d + (k + 1) * dh]
        s = jax.lax.dot_general(q.astype(jnp.bfloat16), kk.astype(jnp.bfloat16),
                                (((1,), (1,)), ((), ())),
                                preferred_element_type=jnp.float32) * scale
        s = s - jnp.max(s, axis=-1, keepdims=True)
        es = jnp.exp(s)
        att = es / jnp.sum(es, axis=-1, keepdims=True)
        outs.append(_mmx(att, v))
    o = _mmx(jnp.concatenate(outs, axis=1), tp['out_w']) + tp['out_b']
    x = _ln(x + o, tp['ln1_g'], tp['ln1_b'])
    ff = _mmx(jnp.maximum(_mmx(x, tp['ff1_w']) + tp['ff1_b'], 0.0),
              tp['ff2_w']) + tp['ff2_b']
    return _ln(x + ff, tp['ln2_g'], tp['ln2_b'])


def _fwd_body(treedef, *refs):
    out_x_ref = refs[-2]
    out_logits_ref = refs[-1]
    vals = [r[:] for r in refs[:-2]]
    p = jax.tree_util.tree_unflatten(treedef, vals)

    x = p['window']
    feats = [x]
    for lp in p['enc']:
        g = _gat(x, lp['W'], lp['A_src'], lp['A_dst'], lp['E'], lp['b'])
        x = jnp.maximum(_ln(g, lp['ln_g'], lp['ln_b']), 0.0)
        feats.append(x)
    bx = feats[-1]
    for tp in p['trans']:
        bx = _tx(bx, tp)
    feats[-1] = bx

    gfeat = jnp.mean(bx, axis=0, keepdims=True)
    hcls = jnp.maximum(_mm(gfeat, p['cls1_w']) + p['cls1_b'], 0.0)
    out_logits_ref[:] = _mm(hcls, p['cls2_w']) + p['cls2_b']

    x = bx
    for i, lp in enumerate(p['dec']):
        g = _gat(x, lp['W'], lp['A_src'], lp['A_dst'], lp['E'], lp['b'])
        x = jnp.maximum(_ln(g, lp['ln_g'], lp['ln_b']), 0.0)
        ef = feats[-(i + 2)]
        aligned = jnp.maximum(
            _ln(_mm(ef, lp['al_w']) + lp['al_b'], lp['al_g'], lp['al_be']), 0.0)
        cat = jnp.concatenate([ef, x], axis=-1)
        gate = jax.nn.sigmoid(
            _mm(jnp.maximum(_mm(cat, lp['g1_w']) + lp['g1_b'], 0.0),
                lp['g2_w']) + lp['g2_b'])
        fin = jnp.concatenate([aligned * gate, x], axis=-1)
        x = jnp.maximum(
            _ln(_mm(fin, lp['fu_w']) + lp['fu_b'], lp['fu_g'], lp['fu_be']), 0.0)

    out_x_ref[:] = x.T


def kernel(window, params):
    n, _ = window.shape

    def row(v):
        return v.reshape(1, -1)

    tree = {'window': window, 'enc': [], 'trans': [], 'dec': []}
    for lp in params['enc']:
        dout = lp['W'].shape[1]
        tree['enc'].append({
            'W': lp['W'],
            'A_src': _attn_expand(lp['a_src']),
            'A_dst': _attn_expand(lp['a_dst']),
            'E': _head_expand(dout),
            'b': row(lp['b']),
            'ln_g': row(lp['ln_g']), 'ln_b': row(lp['ln_b'])})
    for tp in params['trans']:
        tree['trans'].append({
            'in_w': tp['in_w'], 'in_b': row(tp['in_b']),
            'out_w': tp['out_w'], 'out_b': row(tp['out_b']),
            'ln1_g': row(tp['ln1_g']), 'ln1_b': row(tp['ln1_b']),
            'ln2_g': row(tp['ln2_g']), 'ln2_b': row(tp['ln2_b']),
            'ff1_w': tp['ff1_w'], 'ff1_b': row(tp['ff1_b']),
            'ff2_w': tp['ff2_w'], 'ff2_b': row(tp['ff2_b'])})
    for lp in params['dec']:
        dout = lp['W'].shape[1]
        tree['dec'].append({
            'W': lp['W'],
            'A_src': _attn_expand(lp['a_src']),
            'A_dst': _attn_expand(lp['a_dst']),
            'E': _head_expand(dout),
            'b': row(lp['b']),
            'ln_g': row(lp['ln_g']), 'ln_b': row(lp['ln_b']),
            'al_w': lp['al_w'], 'al_b': row(lp['al_b']),
            'al_g': row(lp['al_g']), 'al_be': row(lp['al_be']),
            'g1_w': lp['g1_w'], 'g1_b': row(lp['g1_b']),
            'g2_w': lp['g2_w'], 'g2_b': row(lp['g2_b']),
            'fu_w': lp['fu_w'], 'fu_b': row(lp['fu_b']),
            'fu_g': row(lp['fu_g']), 'fu_be': row(lp['fu_be'])})
    tree['cls1_w'] = params['cls1_w']
    tree['cls1_b'] = row(params['cls1_b'])
    tree['cls2_w'] = params['cls2_w']
    tree['cls2_b'] = row(params['cls2_b'])

    flat, treedef = jax.tree_util.tree_flatten(tree)
    out_ch = params['dec'][-1]['W'].shape[1]

    x_t, logits = pl.pallas_call(
        functools.partial(_fwd_body, treedef),
        out_shape=[
            jax.ShapeDtypeStruct((out_ch, n), jnp.float32),
            jax.ShapeDtypeStruct((1, 2), jnp.float32),
        ],
        compiler_params=pltpu.CompilerParams(
            vmem_limit_bytes=128 * 1024 * 1024),
    )(*flat)
    return (x_t, logits.reshape(2))


# speedup vs baseline: 45.7144x; 1.0030x over previous
import functools

import jax
import jax.numpy as jnp
import numpy as np
from jax.experimental import pallas as pl
from jax.experimental.pallas import tpu as pltpu

_HEADS = 4
_NEG = -1e30


def _attn_expand(a):
    h, dh = a.shape
    eye = jnp.eye(h, dtype=a.dtype)
    return (a[:, :, None] * eye[:, None, :]).reshape(h * dh, h)


def _head_expand(dout):
    return jnp.repeat(jnp.eye(_HEADS, dtype=jnp.float32), dout // _HEADS, axis=1)


def _ln(x, g, b):
    m = jnp.mean(x, axis=-1, keepdims=True)
    v = jnp.mean((x - m) * (x - m), axis=-1, keepdims=True)
    return (x - m) * jax.lax.rsqrt(v + 1e-5) * g + b


def _lrelu(x):
    return jnp.where(x >= 0, x, 0.2 * x)


def _mm(a, b):
    return jnp.dot(a, b, preferred_element_type=jnp.float32)


def _mmx(a, b):
    return jnp.dot(a.astype(jnp.bfloat16), b.astype(jnp.bfloat16),
                   preferred_element_type=jnp.float32)


def _gat(x, W, A_src, A_dst, E, b):
    h = _mm(x, W)
    asrc = _mm(h, A_src)
    adst = _mm(h, A_dst)
    neg = jnp.full((1, _HEADS), _NEG, jnp.float32)
    asrc_m1 = jnp.concatenate([neg, asrc[:-1]], axis=0)
    asrc_p1 = jnp.concatenate([asrc[1:], neg], axis=0)
    e_s = _lrelu(asrc + adst)
    e_m = _lrelu(asrc_m1 + adst)
    e_p = _lrelu(asrc_p1 + adst)
    emax = jnp.maximum(e_s, jnp.maximum(e_m, e_p))
    ex_s = jnp.exp(e_s - emax)
    ex_m = jnp.exp(e_m - emax)
    ex_p = jnp.exp(e_p - emax)
    den = ex_s + ex_m + ex_p + 1e-16
    zrow = jnp.zeros((1, h.shape[1]), jnp.float32)
    h_m1 = jnp.concatenate([zrow, h[:-1]], axis=0)
    h_p1 = jnp.concatenate([h[1:], zrow], axis=0)
    out = (_mm(ex_s / den, E) * h
           + _mm(ex_m / den, E) * h_m1
           + _mm(ex_p / den, E) * h_p1)
    return out + b


def _tx(x, tp):
    n, d = x.shape
    dh = d // _HEADS
    qkv = _mmx(x, tp['in_w']) + tp['in_b']
    scale = 1.0 / np.sqrt(dh)
    outs = []
    for k in range(_HEADS):
        q = qkv[:, k * dh:(k + 1) * dh]
        kk = qkv[:, d + k * dh:d + (k + 1) * dh]
        v = qkv[:, 2 * d + k * dh:2 * d + (k + 1) * dh]
        s = jax.lax.dot_general(q.astype(jnp.bfloat16), kk.astype(jnp.bfloat16),
                                (((1,), (1,)), ((), ())),
                                preferred_element_type=jnp.float32) * scale
        es = jnp.exp(s - jnp.max(s, axis=-1, keepdims=True)).astype(jnp.bfloat16)
        den = jnp.sum(es, axis=-1, keepdims=True, dtype=jnp.float32)
        outs.append(_mm(es, v.astype(jnp.bfloat16)) / den)
    o = _mmx(jnp.concatenate(outs, axis=1), tp['out_w']) + tp['out_b']
    x = _ln(x + o, tp['ln1_g'], tp['ln1_b'])
    ff = _mmx(jnp.maximum(_mmx(x, tp['ff1_w']) + tp['ff1_b'], 0.0),
              tp['ff2_w']) + tp['ff2_b']
    return _ln(x + ff, tp['ln2_g'], tp['ln2_b'])


def _fwd_body(treedef, *refs):
    out_x_ref = refs[-2]
    out_logits_ref = refs[-1]
    vals = [r[:] for r in refs[:-2]]
    p = jax.tree_util.tree_unflatten(treedef, vals)

    x = p['window']
    feats = [x]
    for lp in p['enc']:
        g = _gat(x, lp['W'], lp['A_src'], lp['A_dst'], lp['E'], lp['b'])
        x = jnp.maximum(_ln(g, lp['ln_g'], lp['ln_b']), 0.0)
        feats.append(x)
    bx = feats[-1]
    for tp in p['trans']:
        bx = _tx(bx, tp)
    feats[-1] = bx

    gfeat = jnp.mean(bx, axis=0, keepdims=True)
    hcls = jnp.maximum(_mm(gfeat, p['cls1_w']) + p['cls1_b'], 0.0)
    out_logits_ref[:] = _mm(hcls, p['cls2_w']) + p['cls2_b']

    x = bx
    for i, lp in enumerate(p['dec']):
        g = _gat(x, lp['W'], lp['A_src'], lp['A_dst'], lp['E'], lp['b'])
        x = jnp.maximum(_ln(g, lp['ln_g'], lp['ln_b']), 0.0)
        ef = feats[-(i + 2)]
        aligned = jnp.maximum(
            _ln(_mm(ef, lp['al_w']) + lp['al_b'], lp['al_g'], lp['al_be']), 0.0)
        cat = jnp.concatenate([ef, x], axis=-1)
        gate = jax.nn.sigmoid(
            _mm(jnp.maximum(_mm(cat, lp['g1_w']) + lp['g1_b'], 0.0),
                lp['g2_w']) + lp['g2_b'])
        fin = jnp.concatenate([aligned * gate, x], axis=-1)
        x = jnp.maximum(
            _ln(_mm(fin, lp['fu_w']) + lp['fu_b'], lp['fu_g'], lp['fu_be']), 0.0)

    out_x_ref[:] = x.T


def kernel(window, params):
    n, _ = window.shape

    def row(v):
        return v.reshape(1, -1)

    tree = {'window': window, 'enc': [], 'trans': [], 'dec': []}
    for lp in params['enc']:
        dout = lp['W'].shape[1]
        tree['enc'].append({
            'W': lp['W'],
            'A_src': _attn_expand(lp['a_src']),
            'A_dst': _attn_expand(lp['a_dst']),
            'E': _head_expand(dout),
            'b': row(lp['b']),
            'ln_g': row(lp['ln_g']), 'ln_b': row(lp['ln_b'])})
    bf = jnp.bfloat16
    for tp in params['trans']:
        tree['trans'].append({
            'in_w': tp['in_w'].astype(bf), 'in_b': row(tp['in_b']),
            'out_w': tp['out_w'].astype(bf), 'out_b': row(tp['out_b']),
            'ln1_g': row(tp['ln1_g']), 'ln1_b': row(tp['ln1_b']),
            'ln2_g': row(tp['ln2_g']), 'ln2_b': row(tp['ln2_b']),
            'ff1_w': tp['ff1_w'].astype(bf), 'ff1_b': row(tp['ff1_b']),
            'ff2_w': tp['ff2_w'].astype(bf), 'ff2_b': row(tp['ff2_b'])})
    for lp in params['dec']:
        dout = lp['W'].shape[1]
        tree['dec'].append({
            'W': lp['W'],
            'A_src': _attn_expand(lp['a_src']),
            'A_dst': _attn_expand(lp['a_dst']),
            'E': _head_expand(dout),
            'b': row(lp['b']),
            'ln_g': row(lp['ln_g']), 'ln_b': row(lp['ln_b']),
            'al_w': lp['al_w'], 'al_b': row(lp['al_b']),
            'al_g': row(lp['al_g']), 'al_be': row(lp['al_be']),
            'g1_w': lp['g1_w'], 'g1_b': row(lp['g1_b']),
            'g2_w': lp['g2_w'], 'g2_b': row(lp['g2_b']),
            'fu_w': lp['fu_w'], 'fu_b': row(lp['fu_b']),
            'fu_g': row(lp['fu_g']), 'fu_be': row(lp['fu_be'])})
    tree['cls1_w'] = params['cls1_w']
    tree['cls1_b'] = row(params['cls1_b'])
    tree['cls2_w'] = params['cls2_w']
    tree['cls2_b'] = row(params['cls2_b'])

    flat, treedef = jax.tree_util.tree_flatten(tree)
    out_ch = params['dec'][-1]['W'].shape[1]

    x_t, logits = pl.pallas_call(
        functools.partial(_fwd_body, treedef),
        out_shape=[
            jax.ShapeDtypeStruct((out_ch, n), jnp.float32),
            jax.ShapeDtypeStruct((1, 2), jnp.float32),
        ],
        compiler_params=pltpu.CompilerParams(
            vmem_limit_bytes=128 * 1024 * 1024),
    )(*flat)
    return (x_t, logits.reshape(2))
